# baseline (speedup 1.0000x reference)
import functools

import jax
import jax.numpy as jnp
from jax import lax
from jax.experimental import pallas as pl
from jax.experimental.pallas import tpu as pltpu
from jax.experimental.pallas import tpu_sc as plsc

N = 10000
E = 320000
D = 128
H = 128
G = 64

NC = 2
NS = 16
NW = NC * NS
EW = E // NW
K = 128
NWIN = EW // K
NG = NWIN // 2
REM = EW - NWIN * K

NPAD = 10240
DEGC = NPAD // NS
RT = 624
ZC = 104

def _sc_degree_body(ei_hbm, out_hbm, acc_sh, zbuf, ones_v, didx, didx_r,
                    si_a, si_b, ss_a, ss_b):
    c = lax.axis_index("c")
    s = lax.axis_index("s")
    base = (c * NS + s) * EW
    dst_hbm = ei_hbm
    base = E + base

    def fill(i, _):
        for j in range(8):
            zbuf[pl.ds(i * 128 + j * 16, 16)] = jnp.zeros((16,), jnp.float32)
        return 0

    lax.fori_loop(0, DEGC // 128, fill, 0)

    for j in range(K // 16):
        ones_v[pl.ds(j * 16, 16)] = jnp.ones((16,), jnp.float32)

    pltpu.sync_copy(zbuf, acc_sh.at[pl.ds(s * DEGC, DEGC)])
    plsc.subcore_barrier()

    didx_a = didx.at[0]
    didx_b = didx.at[1]

    def fetch(w, buf, sem):
        pltpu.async_copy(dst_hbm.at[pl.ds(base + w * K, K)], buf, sem)

    def wfetch(w, buf, sem):
        pltpu.make_async_copy(dst_hbm.at[pl.ds(base + w * K, K)], buf,
                              sem).wait()

    def wscat(buf, sem):
        pltpu.make_async_copy(ones_v, acc_sh.at[buf], sem).wait()

    fetch(0, didx_a, si_a)
    fetch(1, didx_b, si_b)

    def group(g, _):
        w0 = 2 * g
        wfetch(w0, didx_a, si_a)
        pltpu.async_copy(ones_v, acc_sh.at[didx_a], ss_a, add=True)

        @pl.when(g + 1 < NG)
        def _():
            wscat(didx_a, ss_a)
            fetch(w0 + 2, didx_a, si_a)

        wfetch(w0 + 1, didx_b, si_b)
        pltpu.async_copy(ones_v, acc_sh.at[didx_b], ss_b, add=True)

        @pl.when(g + 1 < NG)
        def _():
            wscat(didx_b, ss_b)
            fetch(w0 + 3, didx_b, si_b)

        return 0

    lax.fori_loop(0, NG, group, 0)
    wscat(didx_a, ss_a)
    wscat(didx_b, ss_b)
    pltpu.sync_copy(dst_hbm.at[pl.ds(base + NWIN * K, REM)], didx_r)
    pltpu.sync_copy(ones_v.at[pl.ds(0, REM)], acc_sh.at[didx_r], add=True)
    plsc.subcore_barrier()
    pltpu.sync_copy(acc_sh.at[pl.ds(s * DEGC, DEGC)],
                    out_hbm.at[c, pl.ds(s * DEGC, DEGC)])


def _sc_aggregate_body(p_hbm, ei_hbm, out_hbm, acc_sh,
                       rows_a, rows_b, zsrc, sidx_a, sidx_b,
                       didx_a, didx_b, sidx_r, didx_r,
                       sg_a, sg_b, si_a, si_b, ss_a, ss_b):
    c = lax.axis_index("c")
    s = lax.axis_index("s")
    base = (c * NS + s) * EW
    dbase = E + base

    def fill_row(i, _):
        for j in range(H // 16):
            zsrc[i, pl.ds(j * 16, 16)] = jnp.zeros((16,), jnp.float32)
        return 0

    lax.fori_loop(0, ZC, fill_row, 0)

    r0 = s * RT

    def zero_chunk(i, _):
        pltpu.sync_copy(zsrc, acc_sh.at[pl.ds(r0 + i * ZC, ZC)])
        return 0

    lax.fori_loop(0, RT // ZC, zero_chunk, 0)

    @pl.when(s == NS - 1)
    def _():
        pltpu.sync_copy(zsrc.at[pl.ds(0, 16)], acc_sh.at[pl.ds(N - 16, 16)])

    plsc.subcore_barrier()

    def fetch_idx(w, sidx, didx, si):
        pltpu.async_copy(ei_hbm.at[pl.ds(base + w * K, K)], sidx, si)
        pltpu.async_copy(ei_hbm.at[pl.ds(dbase + w * K, K)], didx, si)

    def wait_idx(w, sidx, didx, si):
        pltpu.make_async_copy(ei_hbm.at[pl.ds(base + w * K, K)], sidx, si).wait()
        pltpu.make_async_copy(ei_hbm.at[pl.ds(dbase + w * K, K)], didx, si).wait()

    def wait_gather(rows, sidx, sg):
        pltpu.make_async_copy(p_hbm.at[sidx], rows, sg).wait()

    def wait_scatter(rows, didx, ss):
        pltpu.make_async_copy(rows, acc_sh.at[didx], ss).wait()

    fetch_idx(0, sidx_a, didx_a, si_a)
    fetch_idx(1, sidx_b, didx_b, si_b)
    wait_idx(0, sidx_a, didx_a, si_a)
    pltpu.async_copy(p_hbm.at[sidx_a], rows_a, sg_a)

    def group(g, _):
        w0 = 2 * g
        w1 = w0 + 1
        wait_idx(w1, sidx_b, didx_b, si_b)
        pltpu.async_copy(p_hbm.at[sidx_b], rows_b, sg_b)
        wait_gather(rows_a, sidx_a, sg_a)
        pltpu.async_copy(rows_a, acc_sh.at[didx_a], ss_a, add=True)

        @pl.when(g + 1 < NG)
        def _():
            wait_scatter(rows_a, didx_a, ss_a)
            fetch_idx(w0 + 2, sidx_a, didx_a, si_a)
            wait_idx(w0 + 2, sidx_a, didx_a, si_a)
            pltpu.async_copy(p_hbm.at[sidx_a], rows_a, sg_a)

        wait_gather(rows_b, sidx_b, sg_b)
        pltpu.async_copy(rows_b, acc_sh.at[didx_b], ss_b, add=True)

        @pl.when(g + 1 < NG)
        def _():
            wait_scatter(rows_b, didx_b, ss_b)
            fetch_idx(w0 + 3, sidx_b, didx_b, si_b)

        return 0

    lax.fori_loop(0, NG, group, 0)
    wait_scatter(rows_a, didx_a, ss_a)
    wait_scatter(rows_b, didx_b, ss_b)
    pltpu.sync_copy(ei_hbm.at[pl.ds(base + NWIN * K, REM)], sidx_r)
    pltpu.sync_copy(ei_hbm.at[pl.ds(dbase + NWIN * K, REM)], didx_r)
    rows_r = rows_a.at[pl.ds(0, REM)]
    pltpu.async_copy(p_hbm.at[sidx_r], rows_r, sg_a).wait()
    pltpu.sync_copy(rows_r, acc_sh.at[didx_r], add=True)
    plsc.subcore_barrier()
    pltpu.sync_copy(acc_sh.at[pl.ds(r0, RT)], out_hbm.at[c, pl.ds(r0, RT)])

    @pl.when(s == NS - 1)
    def _():
        pltpu.sync_copy(acc_sh.at[pl.ds(N - 16, 16)],
                        out_hbm.at[c, pl.ds(N - 16, 16)])


@functools.cache
def _sc_kernels():
    mesh = plsc.VectorSubcoreMesh(core_axis_name="c", subcore_axis_name="s",
                                  num_cores=NC, num_subcores=NS)
    sc_degree = pl.kernel(
        _sc_degree_body,
        out_type=jax.ShapeDtypeStruct((NC, NPAD), jnp.float32),
        mesh=mesh,
        scratch_types=[
            pltpu.VMEM_SHARED((NPAD,), jnp.float32),
            pltpu.VMEM((DEGC,), jnp.float32),
            pltpu.VMEM((K,), jnp.float32),
            pltpu.VMEM((2, K), jnp.int32),
            pltpu.VMEM((REM,), jnp.int32),
            pltpu.SemaphoreType.DMA,
            pltpu.SemaphoreType.DMA,
            pltpu.SemaphoreType.DMA,
            pltpu.SemaphoreType.DMA,
        ],
    )
    sc_aggregate = pl.kernel(
        _sc_aggregate_body,
        out_type=jax.ShapeDtypeStruct((NC, N, H), jnp.float32),
        mesh=mesh,
        scratch_types=(
            [pltpu.VMEM_SHARED((N, H), jnp.float32)]
            + [pltpu.VMEM((K, H), jnp.float32)] * 2
            + [pltpu.VMEM((ZC, H), jnp.float32)]
            + [pltpu.VMEM((K,), jnp.int32)] * 4
            + [pltpu.VMEM((REM,), jnp.int32)] * 2
            + [pltpu.SemaphoreType.DMA] * 6
        ),
    )
    return sc_degree, sc_aggregate


_RB = 1000
_GRID = N // _RB

_HIGH = lax.Precision.HIGHEST


def _mm_body(x_ref, w_ref, o_ref):
    o_ref[...] = jnp.dot(x_ref[...], w_ref[...], precision=_HIGH,
                         preferred_element_type=jnp.float32)


def _scale_body(xw_ref, degs_ref, p_ref, dinv_ref):
    deg = degs_ref[0] + degs_ref[1] + 1.0
    dinv = lax.rsqrt(deg)
    p_ref[...] = xw_ref[...] * dinv
    dinv_ref[...] = dinv


def _fuse_body(acc_ref, p_ref, dinv_ref, b_ref, w_ref, pn_ref):
    dinv = dinv_ref[...]
    t = dinv * (acc_ref[0] + acc_ref[1] + p_ref[...]) + b_ref[...]
    h = jnp.maximum(t, 0.0)
    pn_ref[...] = jnp.dot(h, w_ref[...], precision=_HIGH,
                          preferred_element_type=jnp.float32) * dinv


def _final_body(acc_ref, p_ref, dinv_ref, b_ref, batch_ref,
                wl1_ref, bl1_ref, wl2_ref, bl2_ref, o_ref,
                sums, counts):
    i = pl.program_id(0)

    @pl.when(i == 0)
    def _():
        sums[...] = jnp.zeros_like(sums)
        counts[...] = jnp.zeros_like(counts)

    dinv = dinv_ref[...]
    t = dinv * (acc_ref[0] + acc_ref[1] + p_ref[...]) + b_ref[...]
    h = jnp.maximum(t, 0.0)
    bt = batch_ref[0]
    gids = lax.broadcasted_iota(jnp.int32, (G, _RB), 0)
    seg = (gids == bt).astype(jnp.float32)
    sums[...] += jnp.dot(seg, h, precision=_HIGH,
                         preferred_element_type=jnp.float32)
    counts[...] += jnp.sum(seg, axis=1, keepdims=True)

    @pl.when(i == _GRID - 1)
    def _():
        g = sums[...] / jnp.maximum(counts[...], 1.0)
        t1 = jnp.maximum(
            jnp.dot(g, wl1_ref[...], precision=_HIGH,
                    preferred_element_type=jnp.float32) + bl1_ref[...], 0.0)
        o_ref[...] = jnp.dot(t1, wl2_ref[...], precision=_HIGH,
                             preferred_element_type=jnp.float32) + bl2_ref[...]


def _row_spec(shape):
    return pl.BlockSpec((_RB,) + shape[1:], lambda i: (i,) + (0,) * (len(shape) - 1))


def _full(shape):
    return pl.BlockSpec(shape, lambda i: (0,) * len(shape))


_mm = pl.pallas_call(
    _mm_body,
    grid=(_GRID,),
    in_specs=[_row_spec((N, D)), _full((D, H))],
    out_specs=_row_spec((N, H)),
    out_shape=jax.ShapeDtypeStruct((N, H), jnp.float32),
)

_scale = pl.pallas_call(
    _scale_body,
    grid=(_GRID,),
    in_specs=[_row_spec((N, H)),
              pl.BlockSpec((NC, _RB, 1), lambda i: (0, i, 0))],
    out_specs=[_row_spec((N, H)), _row_spec((N, 1))],
    out_shape=[jax.ShapeDtypeStruct((N, H), jnp.float32),
               jax.ShapeDtypeStruct((N, 1), jnp.float32)],
)

_acc_spec = pl.BlockSpec((NC, _RB, H), lambda i: (0, i, 0))

_fuse = pl.pallas_call(
    _fuse_body,
    grid=(_GRID,),
    in_specs=[_acc_spec, _row_spec((N, H)), _row_spec((N, 1)),
              _full((1, H)), _full((H, H))],
    out_specs=_row_spec((N, H)),
    out_shape=jax.ShapeDtypeStruct((N, H), jnp.float32),
)

_final = pl.pallas_call(
    _final_body,
    grid=(_GRID,),
    in_specs=[_acc_spec, _row_spec((N, H)), _row_spec((N, 1)),
              _full((1, H)),
              pl.BlockSpec((1, 1, _RB), lambda i: (i, 0, 0)),
              _full((H, H)), _full((1, H)), _full((H, 1)), _full((1, 1))],
    out_specs=_full((G, 1)),
    out_shape=jax.ShapeDtypeStruct((G, 1), jnp.float32),
    scratch_shapes=[pltpu.VMEM((G, H), jnp.float32),
                    pltpu.VMEM((G, 1), jnp.float32)],
    compiler_params=pltpu.CompilerParams(
        dimension_semantics=("arbitrary",)),
)


def kernel(x, edge_index, batch, Wc0, bc0, Wc1, bc1, Wc2, bc2, Wl1, bl1, Wl2, bl2):
    _sc_degree, _sc_aggregate = _sc_kernels()

    xw0 = _mm(x, Wc0)
    ei_flat = edge_index.reshape(2 * E)
    deg_parts = _sc_degree(ei_flat)
    p0, dinv = _scale(xw0, deg_parts.reshape(NC, NPAD, 1))

    acc0 = _sc_aggregate(p0, ei_flat)
    p1 = _fuse(acc0, p0, dinv, bc0.reshape(1, H), Wc1)

    acc1 = _sc_aggregate(p1, ei_flat)
    p2 = _fuse(acc1, p1, dinv, bc1.reshape(1, H), Wc2)

    acc2 = _sc_aggregate(p2, ei_flat)
    out = _final(acc2, p2, dinv, bc2.reshape(1, H),
                 batch.reshape(_GRID, 1, _RB),
                 Wl1, bl1.reshape(1, H), Wl2, bl2.reshape(1, 1))
    return out.reshape(-1)


# speedup vs baseline: 1.0184x; 1.0184x over previous
import functools

import jax
import jax.numpy as jnp
from jax import lax
from jax.experimental import pallas as pl
from jax.experimental.pallas import tpu as pltpu
from jax.experimental.pallas import tpu_sc as plsc

N = 10000
E = 320000
D = 128
H = 128
G = 64

NC = 2
NS = 16
NW = NC * NS
EW = E // NW
K = 128
NWIN = EW // K
NG = NWIN // 2
REM = EW - NWIN * K

NPAD = 10240
DEGC = NPAD // NS
RT = 624
ZC = 104

def _sc_degree_body(ei_hbm, out_hbm, acc_sh, zbuf, ones_v, didx, didx_r,
                    si_a, si_b, ss_a, ss_b):
    c = lax.axis_index("c")
    s = lax.axis_index("s")
    base = (c * NS + s) * EW
    dst_hbm = ei_hbm
    base = E + base

    def fill(i, _):
        for j in range(8):
            zbuf[pl.ds(i * 128 + j * 16, 16)] = jnp.zeros((16,), jnp.float32)
        return 0

    lax.fori_loop(0, DEGC // 128, fill, 0)

    for j in range(K // 16):
        ones_v[pl.ds(j * 16, 16)] = jnp.ones((16,), jnp.float32)

    pltpu.sync_copy(zbuf, acc_sh.at[pl.ds(s * DEGC, DEGC)])
    plsc.subcore_barrier()

    didx_a = didx.at[0]
    didx_b = didx.at[1]

    def fetch(w, buf, sem):
        pltpu.async_copy(dst_hbm.at[pl.ds(base + w * K, K)], buf, sem)

    def wfetch(w, buf, sem):
        pltpu.make_async_copy(dst_hbm.at[pl.ds(base + w * K, K)], buf,
                              sem).wait()

    def wscat(buf, sem):
        pltpu.make_async_copy(ones_v, acc_sh.at[buf], sem).wait()

    fetch(0, didx_a, si_a)
    fetch(1, didx_b, si_b)

    def group(g, _):
        w0 = 2 * g
        wfetch(w0, didx_a, si_a)
        pltpu.async_copy(ones_v, acc_sh.at[didx_a], ss_a, add=True)

        @pl.when(g + 1 < NG)
        def _():
            wscat(didx_a, ss_a)
            fetch(w0 + 2, didx_a, si_a)

        wfetch(w0 + 1, didx_b, si_b)
        pltpu.async_copy(ones_v, acc_sh.at[didx_b], ss_b, add=True)

        @pl.when(g + 1 < NG)
        def _():
            wscat(didx_b, ss_b)
            fetch(w0 + 3, didx_b, si_b)

        return 0

    lax.fori_loop(0, NG, group, 0)
    wscat(didx_a, ss_a)
    wscat(didx_b, ss_b)
    pltpu.sync_copy(dst_hbm.at[pl.ds(base + NWIN * K, REM)], didx_r)
    pltpu.sync_copy(ones_v.at[pl.ds(0, REM)], acc_sh.at[didx_r], add=True)
    plsc.subcore_barrier()
    pltpu.sync_copy(acc_sh.at[pl.ds(s * DEGC, DEGC)],
                    out_hbm.at[c, pl.ds(s * DEGC, DEGC)])


def _sc_aggregate_body(p_hbm, ei_hbm, out_hbm, acc_sh,
                       rows_a, rows_b, zsrc, sidx_a, sidx_b,
                       didx_a, didx_b, sidx_r, didx_r,
                       sg_a, sg_b, si_a, si_b, ss_a, ss_b):
    c = lax.axis_index("c")
    s = lax.axis_index("s")
    base = (c * NS + s) * EW
    dbase = E + base

    def fill_row(i, _):
        for j in range(H // 16):
            zsrc[i, pl.ds(j * 16, 16)] = jnp.zeros((16,), jnp.float32)
        return 0

    lax.fori_loop(0, ZC, fill_row, 0)

    r0 = s * RT

    def zero_chunk(i, _):
        pltpu.sync_copy(zsrc, acc_sh.at[pl.ds(r0 + i * ZC, ZC)])
        return 0

    lax.fori_loop(0, RT // ZC, zero_chunk, 0)

    @pl.when(s == NS - 1)
    def _():
        pltpu.sync_copy(zsrc.at[pl.ds(0, 16)], acc_sh.at[pl.ds(N - 16, 16)])

    plsc.subcore_barrier()

    def fetch_idx(w, sidx, didx, si):
        pltpu.async_copy(ei_hbm.at[pl.ds(base + w * K, K)], sidx, si)
        pltpu.async_copy(ei_hbm.at[pl.ds(dbase + w * K, K)], didx, si)

    def wait_idx(w, sidx, didx, si):
        pltpu.make_async_copy(ei_hbm.at[pl.ds(base + w * K, K)], sidx, si).wait()
        pltpu.make_async_copy(ei_hbm.at[pl.ds(dbase + w * K, K)], didx, si).wait()

    def wait_gather(rows, sidx, sg):
        pltpu.make_async_copy(p_hbm.at[sidx], rows, sg).wait()

    def wait_scatter(rows, didx, ss):
        pltpu.make_async_copy(rows, acc_sh.at[didx], ss).wait()

    fetch_idx(0, sidx_a, didx_a, si_a)
    fetch_idx(1, sidx_b, didx_b, si_b)
    wait_idx(0, sidx_a, didx_a, si_a)
    pltpu.async_copy(p_hbm.at[sidx_a], rows_a, sg_a)

    def group(g, _):
        w0 = 2 * g
        w1 = w0 + 1
        wait_idx(w1, sidx_b, didx_b, si_b)
        pltpu.async_copy(p_hbm.at[sidx_b], rows_b, sg_b)
        wait_gather(rows_a, sidx_a, sg_a)
        pltpu.async_copy(rows_a, acc_sh.at[didx_a], ss_a, add=True)

        @pl.when(g + 1 < NG)
        def _():
            wait_scatter(rows_a, didx_a, ss_a)
            fetch_idx(w0 + 2, sidx_a, didx_a, si_a)
            wait_idx(w0 + 2, sidx_a, didx_a, si_a)
            pltpu.async_copy(p_hbm.at[sidx_a], rows_a, sg_a)

        wait_gather(rows_b, sidx_b, sg_b)
        pltpu.async_copy(rows_b, acc_sh.at[didx_b], ss_b, add=True)

        @pl.when(g + 1 < NG)
        def _():
            wait_scatter(rows_b, didx_b, ss_b)
            fetch_idx(w0 + 3, sidx_b, didx_b, si_b)

        return 0

    lax.fori_loop(0, NG, group, 0)
    wait_scatter(rows_a, didx_a, ss_a)
    wait_scatter(rows_b, didx_b, ss_b)
    pltpu.sync_copy(ei_hbm.at[pl.ds(base + NWIN * K, REM)], sidx_r)
    pltpu.sync_copy(ei_hbm.at[pl.ds(dbase + NWIN * K, REM)], didx_r)
    rows_r = rows_a.at[pl.ds(0, REM)]
    pltpu.async_copy(p_hbm.at[sidx_r], rows_r, sg_a).wait()
    pltpu.sync_copy(rows_r, acc_sh.at[didx_r], add=True)
    plsc.subcore_barrier()
    pltpu.sync_copy(acc_sh.at[pl.ds(r0, RT)], out_hbm.at[c, pl.ds(r0, RT)])

    @pl.when(s == NS - 1)
    def _():
        pltpu.sync_copy(acc_sh.at[pl.ds(N - 16, 16)],
                        out_hbm.at[c, pl.ds(N - 16, 16)])


@functools.cache
def _sc_kernels():
    mesh = plsc.VectorSubcoreMesh(core_axis_name="c", subcore_axis_name="s",
                                  num_cores=NC, num_subcores=NS)
    sc_degree = pl.kernel(
        _sc_degree_body,
        out_type=jax.ShapeDtypeStruct((NC, NPAD), jnp.float32),
        mesh=mesh,
        scratch_types=[
            pltpu.VMEM_SHARED((NPAD,), jnp.float32),
            pltpu.VMEM((DEGC,), jnp.float32),
            pltpu.VMEM((K,), jnp.float32),
            pltpu.VMEM((2, K), jnp.int32),
            pltpu.VMEM((REM,), jnp.int32),
            pltpu.SemaphoreType.DMA,
            pltpu.SemaphoreType.DMA,
            pltpu.SemaphoreType.DMA,
            pltpu.SemaphoreType.DMA,
        ],
    )
    sc_aggregate = pl.kernel(
        _sc_aggregate_body,
        out_type=jax.ShapeDtypeStruct((NC, N, H), jnp.float32),
        mesh=mesh,
        scratch_types=(
            [pltpu.VMEM_SHARED((N, H), jnp.float32)]
            + [pltpu.VMEM((K, H), jnp.float32)] * 2
            + [pltpu.VMEM((ZC, H), jnp.float32)]
            + [pltpu.VMEM((K,), jnp.int32)] * 4
            + [pltpu.VMEM((REM,), jnp.int32)] * 2
            + [pltpu.SemaphoreType.DMA] * 6
        ),
    )
    return sc_degree, sc_aggregate


_RB = 1000
_GRID = N // _RB

_HIGH = lax.Precision.DEFAULT


def _mm_body(x_ref, w_ref, o_ref):
    o_ref[...] = jnp.dot(x_ref[...], w_ref[...], precision=_HIGH,
                         preferred_element_type=jnp.float32)


def _scale_body(xw_ref, degs_ref, p_ref, dinv_ref):
    deg = degs_ref[0] + degs_ref[1] + 1.0
    dinv = lax.rsqrt(deg)
    p_ref[...] = xw_ref[...] * dinv
    dinv_ref[...] = dinv


def _fuse_body(acc_ref, p_ref, dinv_ref, b_ref, w_ref, pn_ref):
    dinv = dinv_ref[...]
    t = dinv * (acc_ref[0] + acc_ref[1] + p_ref[...]) + b_ref[...]
    h = jnp.maximum(t, 0.0)
    pn_ref[...] = jnp.dot(h, w_ref[...], precision=_HIGH,
                          preferred_element_type=jnp.float32) * dinv


def _final_body(acc_ref, p_ref, dinv_ref, b_ref, batch_ref,
                wl1_ref, bl1_ref, wl2_ref, bl2_ref, o_ref,
                sums, counts):
    i = pl.program_id(0)

    @pl.when(i == 0)
    def _():
        sums[...] = jnp.zeros_like(sums)
        counts[...] = jnp.zeros_like(counts)

    dinv = dinv_ref[...]
    t = dinv * (acc_ref[0] + acc_ref[1] + p_ref[...]) + b_ref[...]
    h = jnp.maximum(t, 0.0)
    bt = batch_ref[0]
    gids = lax.broadcasted_iota(jnp.int32, (G, _RB), 0)
    seg = (gids == bt).astype(jnp.float32)
    sums[...] += jnp.dot(seg, h, precision=_HIGH,
                         preferred_element_type=jnp.float32)
    counts[...] += jnp.sum(seg, axis=1, keepdims=True)

    @pl.when(i == _GRID - 1)
    def _():
        g = sums[...] / jnp.maximum(counts[...], 1.0)
        t1 = jnp.maximum(
            jnp.dot(g, wl1_ref[...], precision=_HIGH,
                    preferred_element_type=jnp.float32) + bl1_ref[...], 0.0)
        o_ref[...] = jnp.dot(t1, wl2_ref[...], precision=_HIGH,
                             preferred_element_type=jnp.float32) + bl2_ref[...]


def _row_spec(shape):
    return pl.BlockSpec((_RB,) + shape[1:], lambda i: (i,) + (0,) * (len(shape) - 1))


def _full(shape):
    return pl.BlockSpec(shape, lambda i: (0,) * len(shape))


_mm = pl.pallas_call(
    _mm_body,
    grid=(_GRID,),
    in_specs=[_row_spec((N, D)), _full((D, H))],
    out_specs=_row_spec((N, H)),
    out_shape=jax.ShapeDtypeStruct((N, H), jnp.float32),
)

_scale = pl.pallas_call(
    _scale_body,
    grid=(_GRID,),
    in_specs=[_row_spec((N, H)),
              pl.BlockSpec((NC, _RB, 1), lambda i: (0, i, 0))],
    out_specs=[_row_spec((N, H)), _row_spec((N, 1))],
    out_shape=[jax.ShapeDtypeStruct((N, H), jnp.float32),
               jax.ShapeDtypeStruct((N, 1), jnp.float32)],
)

_acc_spec = pl.BlockSpec((NC, _RB, H), lambda i: (0, i, 0))

_fuse = pl.pallas_call(
    _fuse_body,
    grid=(_GRID,),
    in_specs=[_acc_spec, _row_spec((N, H)), _row_spec((N, 1)),
              _full((1, H)), _full((H, H))],
    out_specs=_row_spec((N, H)),
    out_shape=jax.ShapeDtypeStruct((N, H), jnp.float32),
)

_final = pl.pallas_call(
    _final_body,
    grid=(_GRID,),
    in_specs=[_acc_spec, _row_spec((N, H)), _row_spec((N, 1)),
              _full((1, H)),
              pl.BlockSpec((1, 1, _RB), lambda i: (i, 0, 0)),
              _full((H, H)), _full((1, H)), _full((H, 1)), _full((1, 1))],
    out_specs=_full((G, 1)),
    out_shape=jax.ShapeDtypeStruct((G, 1), jnp.float32),
    scratch_shapes=[pltpu.VMEM((G, H), jnp.float32),
                    pltpu.VMEM((G, 1), jnp.float32)],
    compiler_params=pltpu.CompilerParams(
        dimension_semantics=("arbitrary",)),
)


def kernel(x, edge_index, batch, Wc0, bc0, Wc1, bc1, Wc2, bc2, Wl1, bl1, Wl2, bl2):
    _sc_degree, _sc_aggregate = _sc_kernels()

    xw0 = _mm(x, Wc0)
    ei_flat = edge_index.reshape(2 * E)
    deg_parts = _sc_degree(ei_flat)
    p0, dinv = _scale(xw0, deg_parts.reshape(NC, NPAD, 1))

    acc0 = _sc_aggregate(p0, ei_flat)
    p1 = _fuse(acc0, p0, dinv, bc0.reshape(1, H), Wc1)

    acc1 = _sc_aggregate(p1, ei_flat)
    p2 = _fuse(acc1, p1, dinv, bc1.reshape(1, H), Wc2)

    acc2 = _sc_aggregate(p2, ei_flat)
    out = _final(acc2, p2, dinv, bc2.reshape(1, H),
                 batch.reshape(_GRID, 1, _RB),
                 Wl1, bl1.reshape(1, H), Wl2, bl2.reshape(1, 1))
    return out.reshape(-1)
